# Initial kernel scaffold; baseline (speedup 1.0000x reference)
import jax
import jax.numpy as jnp
from jax.experimental import pallas as pl


def kernel(left_idx, right_idx, class_emb, pos_left, pos_right):
    raise NotImplementedError("write your pallas kernel here")


# speedup vs baseline: 1.0449x; 1.0449x over previous
import functools

import jax
import jax.numpy as jnp
from jax import lax
from jax.experimental import pallas as pl
from jax.experimental.pallas import tpu as pltpu
from jax.experimental.pallas import tpu_sc as plsc

NUM_CLASSES_ = 100000
D_ = 128
B_ = 16384
NW_ = 32
ROWS_PER_W = B_ // NW_
CHUNK = 128
NCHUNK = ROWS_PER_W // CHUNK
L_ = 16
NJ = D_ // L_


def _rsqrt16(x):
    i = plsc.bitcast(x, jnp.int32)
    i = 0x5F3759DF - (i >> 1)
    y = plsc.bitcast(i, jnp.float32)
    for _ in range(3):
        y = y * (1.5 - 0.5 * x * y * y)
    return y


def _body(left2d, right2d, table, pos_l, pos_r, out,
          idxl_v, idxr_v, bufl, bufr, poslv, posrv, seml, semr):
    wid = lax.axis_index("s") * 2 + lax.axis_index("c")
    base = wid * ROWS_PER_W

    pltpu.sync_copy(left2d.at[pl.ds(wid * NCHUNK, NCHUNK)], idxl_v)
    pltpu.sync_copy(right2d.at[pl.ds(wid * NCHUNK, NCHUNK)], idxr_v)
    pltpu.sync_copy(pos_l, poslv)
    pltpu.sync_copy(pos_r, posrv)

    posv = [poslv[pl.ds(j * L_, L_)] + posrv[pl.ds(j * L_, L_)]
            for j in range(NJ)]

    for c in range(NCHUNK):
        cl = pltpu.async_copy(table.at[idxl_v.at[c]], bufl, seml)
        cr = pltpu.async_copy(table.at[idxr_v.at[c]], bufr, semr)
        cl.wait()
        cr.wait()

        def row(r, carry):
            e = [bufl[r, pl.ds(j * L_, L_)] + bufr[r, pl.ds(j * L_, L_)]
                 + posv[j] for j in range(NJ)]
            ss = e[0] * e[0]
            for j in range(1, NJ):
                ss = ss + e[j] * e[j]
            tot = jnp.sum(ss)
            tv = jnp.broadcast_to(tot, (L_,))
            rinv = jnp.minimum(_rsqrt16(tv), 1e12)
            for j in range(NJ):
                bufl[r, pl.ds(j * L_, L_)] = e[j] * rinv
            return carry

        lax.fori_loop(0, CHUNK, row, 0)
        pltpu.sync_copy(bufl, out.at[pl.ds(base + c * CHUNK, CHUNK)])


def kernel(left_idx, right_idx, class_emb, pos_left, pos_right):
    left2d = left_idx.reshape(B_ // CHUNK, CHUNK).astype(jnp.int32)
    right2d = right_idx.reshape(B_ // CHUNK, CHUNK).astype(jnp.int32)
    k = pl.kernel(
        _body,
        out_type=jax.ShapeDtypeStruct((B_, D_), jnp.float32),
        mesh=plsc.VectorSubcoreMesh(core_axis_name="c", subcore_axis_name="s"),
        compiler_params=pltpu.CompilerParams(needs_layout_passes=False),
        scratch_types=[
            pltpu.VMEM((NCHUNK, CHUNK), jnp.int32),
            pltpu.VMEM((NCHUNK, CHUNK), jnp.int32),
            pltpu.VMEM((CHUNK, D_), jnp.float32),
            pltpu.VMEM((CHUNK, D_), jnp.float32),
            pltpu.VMEM((D_,), jnp.float32),
            pltpu.VMEM((D_,), jnp.float32),
            pltpu.SemaphoreType.DMA,
            pltpu.SemaphoreType.DMA,
        ],
    )
    return k(left2d, right2d, class_emb, pos_left, pos_right)


# speedup vs baseline: 1.4898x; 1.4258x over previous
import jax
import jax.numpy as jnp
from jax import lax
from jax.experimental import pallas as pl
from jax.experimental.pallas import tpu as pltpu
from jax.experimental.pallas import tpu_sc as plsc

NUM_CLASSES_ = 100000
D_ = 128
B_ = 16384
NW_ = 32
ROWS_PER_W = B_ // NW_
CHUNK = 128
NCHUNK = ROWS_PER_W // CHUNK
L_ = 16
NJ = D_ // L_


def _rsqrt16(x):
    i = plsc.bitcast(x, jnp.int32)
    i = 0x5F3759DF - (i >> 1)
    y = plsc.bitcast(i, jnp.float32)
    for _ in range(3):
        y = y * (1.5 - 0.5 * x * y * y)
    return y


def _compute_chunk(bl, br, ob, posv):
    @plsc.parallel_loop(0, CHUNK, unroll=4)
    def row(r):
        e = [bl[r, pl.ds(j * L_, L_)] + br[r, pl.ds(j * L_, L_)] + posv[j]
             for j in range(NJ)]
        ss = e[0] * e[0]
        for j in range(1, NJ):
            ss = ss + e[j] * e[j]
        tot = jnp.sum(ss)
        tv = jnp.broadcast_to(tot, (L_,))
        rinv = jnp.minimum(_rsqrt16(tv), 1e12)
        for j in range(NJ):
            ob[r, pl.ds(j * L_, L_)] = e[j] * rinv


def _body(left2d, right2d, table, pos_l, pos_r, out,
          idxl_v, idxr_v, bufl0, bufl1, bufr0, bufr1, obuf0, obuf1,
          poslv, posrv, gl0, gl1, gr0, gr1, so0, so1):
    bufl = (bufl0, bufl1)
    bufr = (bufr0, bufr1)
    obuf = (obuf0, obuf1)
    gl = (gl0, gl1)
    gr = (gr0, gr1)
    so = (so0, so1)

    wid = lax.axis_index("s") * 2 + lax.axis_index("c")
    base = wid * ROWS_PER_W

    pltpu.sync_copy(left2d.at[pl.ds(wid * NCHUNK, NCHUNK)], idxl_v)
    pltpu.sync_copy(right2d.at[pl.ds(wid * NCHUNK, NCHUNK)], idxr_v)
    pltpu.sync_copy(pos_l, poslv)
    pltpu.sync_copy(pos_r, posrv)

    posv = [poslv[pl.ds(j * L_, L_)] + posrv[pl.ds(j * L_, L_)]
            for j in range(NJ)]

    def issue_gather(c):
        b = c & 1
        cl = pltpu.async_copy(table.at[idxl_v.at[c]], bufl[b], gl[b])
        cr = pltpu.async_copy(table.at[idxr_v.at[c]], bufr[b], gr[b])
        return cl, cr

    descs = [None] * NCHUNK
    odesc = [None] * NCHUNK
    descs[0] = issue_gather(0)
    for c in range(NCHUNK):
        b = c & 1
        if c + 1 < NCHUNK:
            descs[c + 1] = issue_gather(c + 1)
        descs[c][0].wait()
        descs[c][1].wait()
        if c >= 2:
            odesc[c - 2].wait()
        _compute_chunk(bufl[b], bufr[b], obuf[b], posv)
        odesc[c] = pltpu.async_copy(
            obuf[b], out.at[pl.ds(base + c * CHUNK, CHUNK)], so[b])
    odesc[NCHUNK - 2].wait()
    odesc[NCHUNK - 1].wait()


def kernel(left_idx, right_idx, class_emb, pos_left, pos_right):
    left2d = left_idx.reshape(B_ // CHUNK, CHUNK).astype(jnp.int32)
    right2d = right_idx.reshape(B_ // CHUNK, CHUNK).astype(jnp.int32)
    k = pl.kernel(
        _body,
        out_type=jax.ShapeDtypeStruct((B_, D_), jnp.float32),
        mesh=plsc.VectorSubcoreMesh(core_axis_name="c", subcore_axis_name="s"),
        compiler_params=pltpu.CompilerParams(needs_layout_passes=False),
        scratch_types=[
            pltpu.VMEM((NCHUNK, CHUNK), jnp.int32),
            pltpu.VMEM((NCHUNK, CHUNK), jnp.int32),
            pltpu.VMEM((CHUNK, D_), jnp.float32),
            pltpu.VMEM((CHUNK, D_), jnp.float32),
            pltpu.VMEM((CHUNK, D_), jnp.float32),
            pltpu.VMEM((CHUNK, D_), jnp.float32),
            pltpu.VMEM((CHUNK, D_), jnp.float32),
            pltpu.VMEM((CHUNK, D_), jnp.float32),
            pltpu.VMEM((D_,), jnp.float32),
            pltpu.VMEM((D_,), jnp.float32),
            pltpu.SemaphoreType.DMA,
            pltpu.SemaphoreType.DMA,
            pltpu.SemaphoreType.DMA,
            pltpu.SemaphoreType.DMA,
            pltpu.SemaphoreType.DMA,
            pltpu.SemaphoreType.DMA,
        ],
    )
    return k(left2d, right2d, class_emb, pos_left, pos_right)
